# baseline (speedup 1.0000x reference)
import numpy as np

import jax
import jax.numpy as jnp
from jax.experimental import pallas as pl
from jax.experimental.pallas import tpu as pltpu


def _fused_kernel(x_ref, w0_ref, w1s_ref, b1_ref, t2_ref, b2_ref,
                  w1_ref, bf1_ref, w2_ref, bf2_ref, o_ref):
    f32 = jnp.float32
    bf16 = jnp.bfloat16
    blk = o_ref.shape[0]

    xf = x_ref[0].astype(bf16)
    a0 = jnp.dot(xf, w0_ref[...], preferred_element_type=f32)
    a1 = jnp.dot(xf, w1s_ref[...], preferred_element_type=f32)
    n = 14 * 256
    pooled = jnp.maximum(
        jnp.maximum(a0[:, :n], a0[:, n:]),
        jnp.maximum(a1[:, :n], a1[:, n:]))
    p1f = jnp.maximum(pooled.astype(bf16) + b1_ref[...], bf16(0.0))
    p1 = jnp.concatenate([p1f[:, 256 * hh:256 * (hh + 1)] for hh in range(14)],
                         axis=0)

    z2 = jnp.zeros((blk, 256), bf16)
    up2 = jnp.concatenate([z2, p1[:-blk]], axis=0)
    dn2 = jnp.concatenate([p1[blk:], z2], axis=0)
    z = jnp.concatenate([up2, p1, dn2], axis=1)
    b = jnp.dot(z, t2_ref[...], preferred_element_type=f32)
    b3 = b.astype(bf16).reshape(7, 2 * blk, 512)
    mp2 = jnp.maximum(
        jnp.maximum(b3[:, :blk, :256], b3[:, :blk, 256:]),
        jnp.maximum(b3[:, blk:, :256], b3[:, blk:, 256:]))
    p2 = jnp.maximum(mp2.reshape(7 * blk, 256) + b2_ref[...], bf16(0.0))

    h = jnp.dot(p2[:blk], w1_ref[:256], preferred_element_type=f32)
    for hh in range(1, 7):
        h += jnp.dot(p2[hh * blk:(hh + 1) * blk],
                     w1_ref[hh * 256:(hh + 1) * 256],
                     preferred_element_type=f32)
    h = jnp.maximum(h + bf1_ref[...], 0.0).astype(bf16)
    o_ref[...] = jnp.dot(h, w2_ref[...], preferred_element_type=f32) + bf2_ref[...]


def _stage1_selectors():
    es = []
    for dh in (0, 1):
        e = np.zeros((3, 28, 14), np.float32)
        for kh in range(3):
            for hp in range(14):
                r = 2 * hp + dh + kh - 1
                if 0 <= r < 28:
                    e[kh, r, hp] = 1.0
        es.append(e)
    return es


def kernel(x, t1e, t1o, b1r, t2e, t2o, b2r, w1fc, b1fc, w2fc, b2fc):
    B = x.shape[0]
    blk = 128
    nb = -(-B // blk)
    Bp = nb * blk
    xs = x[:, 0]
    if Bp != B:
        xs = jnp.pad(xs, ((0, Bp - B), (0, 0), (0, 0)))
    xt = xs.reshape(nb, blk, 784)
    bf16 = jnp.bfloat16
    t1c = jnp.concatenate([t1e.reshape(90, 256), t1o.reshape(90, 256)], axis=1)
    t = t1c.reshape(3, 30, 512)[:, 1:29, :].reshape(3, 28, 2, 256)
    e0, e1 = _stage1_selectors()
    w0 = jnp.einsum("krh,kwpc->rwphc", e0, t).reshape(784, 7168).astype(bf16)
    w1s = jnp.einsum("krh,kwpc->rwphc", e1, t).reshape(784, 7168).astype(bf16)
    b1f = jnp.tile(b1r, (1, 14)).astype(bf16)
    t2c = jnp.concatenate([t2e.reshape(768, 256), t2o.reshape(768, 256)],
                          axis=1).astype(bf16)
    b2c = b2r.astype(bf16)
    w1b = w1fc.astype(bf16)
    w2b = w2fc.astype(bf16)
    out = pl.pallas_call(
        _fused_kernel,
        out_shape=jax.ShapeDtypeStruct((Bp, 128), jnp.float32),
        grid=(nb,),
        in_specs=[
            pl.BlockSpec((1, blk, 784), lambda i: (i, 0, 0)),
            pl.BlockSpec((784, 7168), lambda i: (0, 0)),
            pl.BlockSpec((784, 7168), lambda i: (0, 0)),
            pl.BlockSpec((1, 3584), lambda i: (0, 0)),
            pl.BlockSpec((768, 512), lambda i: (0, 0)),
            pl.BlockSpec((1, 256), lambda i: (0, 0)),
            pl.BlockSpec((1792, 128), lambda i: (0, 0)),
            pl.BlockSpec((1, 128), lambda i: (0, 0)),
            pl.BlockSpec((128, 128), lambda i: (0, 0)),
            pl.BlockSpec((1, 128), lambda i: (0, 0)),
        ],
        out_specs=pl.BlockSpec((blk, 128), lambda i: (i, 0)),
        compiler_params=pltpu.CompilerParams(dimension_semantics=("parallel",)),
    )(xt, w0, w1s, b1f, t2c, b2c, w1b, b1fc, w2b, b2fc)
    return out[:B, :10]


# speedup vs baseline: 1.8831x; 1.8831x over previous
import jax
import jax.numpy as jnp
from jax.experimental import pallas as pl
from jax.experimental.pallas import tpu as pltpu


def _fused_kernel(x_ref, t1_ref, b1_ref, t2_ref, b2_ref,
                  w1_ref, bf1_ref, w2_ref, bf2_ref, o_ref):
    f32 = jnp.float32
    bf16 = jnp.bfloat16
    blk = o_ref.shape[0]
    m1 = 28 * blk

    xb = x_ref[...]
    zr = jnp.zeros((blk, 28), f32)
    up = jnp.concatenate([zr, xb[:-blk]], axis=0)
    dn = jnp.concatenate([xb[blk:], zr], axis=0)
    z1 = jnp.zeros((m1, 1), f32)
    z2 = jnp.zeros((m1, 2), f32)
    y = jnp.concatenate([z1, up, z2, xb, z2, dn, z1], axis=1)
    a = jnp.dot(y, t1_ref[...], preferred_element_type=f32)
    a3 = a.astype(bf16).reshape(14, 2 * blk, 512)
    mp = jnp.maximum(
        jnp.maximum(a3[:, :blk, :256], a3[:, :blk, 256:]),
        jnp.maximum(a3[:, blk:, :256], a3[:, blk:, 256:]))
    p1 = jnp.maximum(mp.reshape(14 * blk, 256) + b1_ref[...], bf16(0.0))

    zb = jnp.zeros((blk, 256), bf16)
    up2 = jnp.concatenate([zb, p1[:-blk]], axis=0)
    dn2 = jnp.concatenate([p1[blk:], zb], axis=0)
    z = jnp.concatenate([up2, p1, dn2], axis=1)
    b = jnp.dot(z, t2_ref[...], preferred_element_type=f32)
    b3 = b.astype(bf16).reshape(7, 2 * blk, 512)
    mp2 = jnp.maximum(
        jnp.maximum(b3[:, :blk, :256], b3[:, :blk, 256:]),
        jnp.maximum(b3[:, blk:, :256], b3[:, blk:, 256:]))
    p2 = jnp.maximum(mp2.reshape(7 * blk, 256) + b2_ref[...], bf16(0.0))

    h = jnp.dot(p2[:blk], w1_ref[:256], preferred_element_type=f32)
    for hh in range(1, 7):
        h += jnp.dot(p2[hh * blk:(hh + 1) * blk],
                     w1_ref[hh * 256:(hh + 1) * 256],
                     preferred_element_type=f32)
    h = jnp.maximum(h + bf1_ref[...], 0.0).astype(bf16)
    o_ref[...] = jnp.dot(h, w2_ref[...], preferred_element_type=f32) + bf2_ref[...]


def kernel(x, t1e, t1o, b1r, t2e, t2o, b2r, w1fc, b1fc, w2fc, b2fc):
    B = x.shape[0]
    blk = 128
    nb = -(-B // blk)
    Bp = nb * blk
    xs = x[:, 0]
    if Bp != B:
        xs = jnp.pad(xs, ((0, Bp - B), (0, 0), (0, 0)))
    xt = xs.reshape(nb, blk, 28, 28).transpose(0, 2, 1, 3).reshape(nb * 28 * blk, 28)
    bf16 = jnp.bfloat16
    t1c = jnp.concatenate([t1e.reshape(90, 256), t1o.reshape(90, 256)], axis=1)
    t2c = jnp.concatenate([t2e.reshape(768, 256), t2o.reshape(768, 256)],
                          axis=1).astype(bf16)
    b1c = b1r.astype(bf16)
    b2c = b2r.astype(bf16)
    w1b = w1fc.astype(bf16)
    w2b = w2fc.astype(bf16)
    out = pl.pallas_call(
        _fused_kernel,
        out_shape=jax.ShapeDtypeStruct((Bp, 128), jnp.float32),
        grid=(nb,),
        in_specs=[
            pl.BlockSpec((28 * blk, 28), lambda i: (i, 0)),
            pl.BlockSpec((90, 512), lambda i: (0, 0)),
            pl.BlockSpec((1, 256), lambda i: (0, 0)),
            pl.BlockSpec((768, 512), lambda i: (0, 0)),
            pl.BlockSpec((1, 256), lambda i: (0, 0)),
            pl.BlockSpec((1792, 128), lambda i: (0, 0)),
            pl.BlockSpec((1, 128), lambda i: (0, 0)),
            pl.BlockSpec((128, 128), lambda i: (0, 0)),
            pl.BlockSpec((1, 128), lambda i: (0, 0)),
        ],
        out_specs=pl.BlockSpec((blk, 128), lambda i: (i, 0)),
        compiler_params=pltpu.CompilerParams(dimension_semantics=("parallel",)),
    )(xt, t1c, b1c, t2c, b2c, w1b, b1fc, w2b, b2fc)
    return out[:B, :10]
